# baseline (speedup 1.0000x reference)
import functools

import jax
import jax.numpy as jnp
from jax import lax
from jax.experimental import pallas as pl
from jax.experimental.pallas import tpu as pltpu
from jax.experimental.pallas import tpu_sc as plsc

B, S = 4096, 200
D = 64
DP = 128
NW = 32
SEQ_PER_W = B // NW
G = 4
NGRP = SEQ_PER_W // G


def _make_kernel():
    mesh = plsc.VectorSubcoreMesh(core_axis_name="c", subcore_axis_name="s")

    @functools.partial(
        pl.kernel,
        out_type=jax.ShapeDtypeStruct((B, S, D), jnp.float32),
        mesh=mesh,
        scratch_types=[
            pltpu.VMEM((SEQ_PER_W, S), jnp.int32),
            pltpu.VMEM((G, S, DP), jnp.float32),
            pltpu.SemaphoreType.DMA((G,)),
            pltpu.SemaphoreType.DMA,
        ],
        compiler_params=pltpu.CompilerParams(use_tc_tiling_on_sc=False),
    )
    def emb(tid_hbm, table_hbm, out_hbm, idx_v, rows_v, gsem, wsem):
        wid = lax.axis_index("s") * 2 + lax.axis_index("c")
        seq0 = wid * SEQ_PER_W
        pltpu.sync_copy(tid_hbm.at[pl.ds(seq0, SEQ_PER_W)], idx_v)

        def body(grp, _):
            r0 = grp * G
            gathers = [
                pltpu.async_copy(
                    table_hbm.at[idx_v.at[r0 + b]], rows_v.at[b], gsem.at[b]
                )
                for b in range(G)
            ]
            wbs = []
            for b in range(G):
                gathers[b].wait()
                wbs.append(
                    pltpu.async_copy(
                        rows_v.at[b, :, pl.ds(0, D)],
                        out_hbm.at[seq0 + r0 + b],
                        wsem,
                    )
                )
            for wb in wbs:
                wb.wait()
            return ()

        lax.fori_loop(0, NGRP, body, ())

    return emb


_emb = _make_kernel()


@jax.jit
def kernel(token_ids, weight):
    wp = jnp.pad(weight, ((0, 0), (0, DP - D)))
    return _emb(token_ids, wp)


# speedup vs baseline: 1.0735x; 1.0735x over previous
import functools

import jax
import jax.numpy as jnp
from jax import lax
from jax.experimental import pallas as pl
from jax.experimental.pallas import tpu as pltpu
from jax.experimental.pallas import tpu_sc as plsc

B, S = 4096, 200
D = 64
NW = 32
SEQ_PER_W = B // NW
G = 8
NGRP = SEQ_PER_W // G


def _make_kernel():
    mesh = plsc.VectorSubcoreMesh(core_axis_name="c", subcore_axis_name="s")

    @functools.partial(
        pl.kernel,
        out_type=jax.ShapeDtypeStruct((B, S, D), jnp.float32),
        mesh=mesh,
        scratch_types=[
            pltpu.VMEM((SEQ_PER_W, S), jnp.int32),
            pltpu.VMEM((G, S, D), jnp.float32),
            pltpu.SemaphoreType.DMA((G,)),
            pltpu.SemaphoreType.DMA,
        ],
        compiler_params=pltpu.CompilerParams(use_tc_tiling_on_sc=False),
    )
    def emb(tid_hbm, table_hbm, out_hbm, idx_v, rows_v, gsem, wsem):
        wid = lax.axis_index("s") * 2 + lax.axis_index("c")
        seq0 = wid * SEQ_PER_W
        pltpu.sync_copy(tid_hbm.at[pl.ds(seq0, SEQ_PER_W)], idx_v)

        def body(grp, _):
            r0 = grp * G
            gathers = [
                pltpu.async_copy(
                    table_hbm.at[idx_v.at[r0 + b]], rows_v.at[b], gsem.at[b]
                )
                for b in range(G)
            ]
            wbs = []
            for b in range(G):
                gathers[b].wait()
                wbs.append(
                    pltpu.async_copy(
                        rows_v.at[b], out_hbm.at[seq0 + r0 + b], wsem
                    )
                )
            for wb in wbs:
                wb.wait()
            return ()

        lax.fori_loop(0, NGRP, body, ())

    return emb


_emb = _make_kernel()


@jax.jit
def kernel(token_ids, weight):
    return _emb(token_ids, weight)
